# baseline (speedup 1.0000x reference)
import functools

import jax
import jax.numpy as jnp
from jax import lax
from jax.experimental import pallas as pl
from jax.experimental.pallas import tpu as pltpu
from jax.experimental.pallas import tpu_sc as plsc

_EPS = 1e-5
_K = 8


def _fps_body(pc_ref, perm_ref):
    x = pc_ref[0]
    y = pc_ref[1]
    z = pc_ref[2]
    bb, nn = x.shape
    lane = lax.broadcasted_iota(jnp.int32, (bb, nn), 1)
    boff = lax.broadcasted_iota(jnp.int32, (bb, nn), 0) * nn

    lane128 = lax.broadcasted_iota(jnp.int32, (bb, 128), 1)

    def body(j, st):
        dists, cperm = st
        far = jnp.argmax(dists, axis=1, keepdims=True).astype(jnp.int32)
        cperm = jnp.where(lane128 == j, far, cperm)
        sel = lane == far
        cx = jnp.sum(jnp.where(sel, x, 0.0), axis=1, keepdims=True)
        cy = jnp.sum(jnp.where(sel, y, 0.0), axis=1, keepdims=True)
        cz = jnp.sum(jnp.where(sel, z, 0.0), axis=1, keepdims=True)
        dx = x - cx
        dy = y - cy
        dz = z - cz
        d = dx * dx + dy * dy + dz * dz
        dists = jnp.minimum(dists, d)
        return dists, cperm

    def outer(c, st):
        dists, cperm = st
        dists, cperm = lax.fori_loop(0, 128, body, (dists, cperm))
        perm_ref[:, pl.ds(c * 128, 128)] = cperm
        return dists, cperm

    dists0 = (nn * 1.0 - lane.astype(jnp.float32)) * 1e6
    cperm0 = lane128 + lax.broadcasted_iota(jnp.int32, (bb, 128), 0)
    lax.fori_loop(0, nn // 128, outer, (dists0, cperm0))
    perm_ref[...] = perm_ref[...] + boff


def _fps(pc3, interpret=False):
    _, bb, nn = pc3.shape
    return pl.pallas_call(
        _fps_body,
        out_shape=jax.ShapeDtypeStruct((bb, nn), jnp.int32),
        interpret=interpret,
    )(pc3)


_RB = 256


def _knn_body(pc_ref, cen_ref, out_ref):
    b = pl.program_id(0)
    nn = pc_ref.shape[2]
    xyz = pc_ref[0]
    xj = xyz[0:1, :]
    yj = xyz[1:2, :]
    zj = xyz[2:3, :]
    cen = cen_ref[0]
    cx = cen[:, 0:1]
    cy = cen[:, 1:2]
    cz = cen[:, 2:3]
    dx = cx - xj
    dy = cy - yj
    dz = cz - zj
    d = dx * dx + dy * dy + dz * dz
    lane = lax.broadcasted_iota(jnp.int32, (d.shape[0], nn), 1)
    cols = []
    for _ in range(_K):
        m = jnp.min(d, axis=1, keepdims=True)
        j = jnp.min(jnp.where(d == m, lane, nn), axis=1, keepdims=True)
        cols.append(j)
        d = jnp.where(lane == j, jnp.inf, d)
    out_ref[0] = jnp.concatenate(cols, axis=1) + b * nn


def _knn(pc, xyzr, interpret=False):
    bb, _, nn = pc.shape
    return pl.pallas_call(
        _knn_body,
        grid=(bb, nn // _RB),
        in_specs=[
            pl.BlockSpec((1, 3, nn), lambda b, r: (b, 0, 0)),
            pl.BlockSpec((1, _RB, 3), lambda b, r: (b, r, 0)),
        ],
        out_specs=pl.BlockSpec((1, _RB, _K), lambda b, r: (b, r, 0)),
        out_shape=jax.ShapeDtypeStruct((bb, nn, _K), jnp.int32),
        interpret=interpret,
    )(pc, xyzr)


def _sc_gather(tables, idx2d):
    info = plsc.get_sparse_core_info()
    nw = info.num_cores * info.num_subcores
    nchunks_total, ch = idx2d.shape
    m = nchunks_total * ch
    nch = nchunks_total // nw
    ntab = len(tables)
    mesh = plsc.VectorSubcoreMesh(core_axis_name="c", subcore_axis_name="s")
    out_types = tuple(
        jax.ShapeDtypeStruct((m, t.shape[1]), t.dtype) for t in tables
    )
    scratch = [pltpu.VMEM((nch, ch), jnp.int32)]
    for t in tables:
        scratch.append(pltpu.VMEM((ch, t.shape[1]), t.dtype))
    scratch.append(pltpu.SemaphoreType.DMA)

    def body(*refs):
        tab_refs = refs[:ntab]
        idx_ref = refs[ntab]
        out_refs = refs[ntab + 1:2 * ntab + 1]
        idx_v = refs[2 * ntab + 1]
        row_bufs = refs[2 * ntab + 2:2 * ntab + 2 + ntab]
        sem = refs[-1]
        wid = lax.axis_index("s") * info.num_cores + lax.axis_index("c")
        base_chunk = wid * nch
        pltpu.sync_copy(idx_ref.at[pl.ds(base_chunk, nch)], idx_v)
        for c in range(nch):
            for t in range(ntab):
                pltpu.async_copy(
                    tab_refs[t].at[idx_v.at[c]], row_bufs[t], sem
                ).wait()
                pltpu.sync_copy(
                    row_bufs[t],
                    out_refs[t].at[pl.ds((base_chunk + c) * ch, ch)],
                )

    fn = pl.kernel(body, mesh=mesh, out_type=out_types, scratch_types=scratch)
    out = fn(*tables, idx2d)
    return out if isinstance(out, (tuple, list)) else (out,)


def _dot(a, b):
    return lax.dot_general(
        a, b, (((1,), (0,)), ((), ())), preferred_element_type=jnp.float32
    )


def _inorm(xx):
    m = jnp.mean(xx, axis=0, keepdims=True)
    c = xx - m
    v = jnp.mean(c * c, axis=0, keepdims=True)
    r = 1.0 / jnp.sqrt(v + _EPS)
    return c * r


def _maxpool_k(y, npts):
    m = y[0:npts, :]
    for k in range(1, _K):
        m = jnp.maximum(m, y[k * npts:(k + 1) * npts, :])
    return m


def _mlp1_body(gf_ref, gx_ref, rep_ref, w3_ref, wf_ref, b1_ref,
               w2_ref, b2_ref, w3c_ref, b3_ref, out_ref):
    gx = gx_ref[0] - rep_ref[0]
    y = _dot(gx, w3_ref[...]) + _dot(gf_ref[0], wf_ref[...]) + b1_ref[...]
    y = jnp.maximum(_inorm(y), 0.0)
    y = _dot(y, w2_ref[...]) + b2_ref[...]
    y = jnp.maximum(_inorm(y), 0.0)
    y = _dot(y, w3c_ref[...]) + b3_ref[...]
    y = jnp.maximum(_inorm(y), 0.0)
    out_ref[0] = _maxpool_k(y, out_ref.shape[1])


def _mlp1(gf, gx, rep, w3, wf, b1, w2, b2, w3c, b3, interpret=False):
    bb, mk, dd = gf.shape
    npts = mk // _K
    wspec = lambda shp: pl.BlockSpec(shp, lambda b: tuple(0 for _ in shp))
    return pl.pallas_call(
        _mlp1_body,
        grid=(bb,),
        in_specs=[
            pl.BlockSpec((1, mk, dd), lambda b: (b, 0, 0)),
            pl.BlockSpec((1, mk, 16), lambda b: (b, 0, 0)),
            pl.BlockSpec((1, mk, 16), lambda b: (b, 0, 0)),
            wspec(w3.shape), wspec(wf.shape), wspec(b1.shape),
            wspec(w2.shape), wspec(b2.shape), wspec(w3c.shape), wspec(b3.shape),
        ],
        out_specs=pl.BlockSpec((1, npts, dd), lambda b: (b, 0, 0)),
        out_shape=jax.ShapeDtypeStruct((bb, npts, dd), jnp.float32),
        interpret=interpret,
    )(gf, gx, rep, w3, wf, b1, w2, b2, w3c, b3)


def _mlp2_body(g2_ref, gx_ref, rep_ref, w3_ref, wf_ref, b_ref, out_ref):
    gx = gx_ref[0] - rep_ref[0]
    y = _dot(gx, w3_ref[...]) + _dot(g2_ref[0], wf_ref[...]) + b_ref[...]
    y = _inorm(y)
    out_ref[0] = _maxpool_k(y, out_ref.shape[1])


def _mlp2(g2, gx, rep, w3, wf, b, interpret=False):
    bb, mk, dd = g2.shape
    npts = mk // _K
    wspec = lambda shp: pl.BlockSpec(shp, lambda b: tuple(0 for _ in shp))
    return pl.pallas_call(
        _mlp2_body,
        grid=(bb,),
        in_specs=[
            pl.BlockSpec((1, mk, dd), lambda b: (b, 0, 0)),
            pl.BlockSpec((1, mk, 16), lambda b: (b, 0, 0)),
            pl.BlockSpec((1, mk, 16), lambda b: (b, 0, 0)),
            wspec(w3.shape), wspec(wf.shape), wspec(b.shape),
        ],
        out_specs=pl.BlockSpec((1, npts, dd), lambda b: (b, 0, 0)),
        out_shape=jax.ShapeDtypeStruct((bb, npts, dd), jnp.float32),
        interpret=interpret,
    )(g2, gx, rep, w3, wf, b)


def kernel(pc, feature, W1a, b1a, W1b, b1b, W1c, b1c, W2a, b2a):
    bb, _, nn = pc.shape
    dd = feature.shape[1]

    pc3 = jnp.transpose(pc, (1, 0, 2))
    xyzr = jnp.transpose(pc, (0, 2, 1))
    xyz16 = jnp.pad(xyzr, ((0, 0), (0, 0), (0, 13))).reshape(bb * nn, 16)
    xyz128 = jnp.pad(xyzr, ((0, 0), (0, 0), (0, 125))).reshape(bb * nn, 128)
    featr = jnp.transpose(feature, (0, 2, 1)).reshape(bb * nn, dd)

    perm_g = _fps(pc3)
    idx = _knn(pc, xyzr)
    idx_km = jnp.transpose(idx, (0, 2, 1))
    idx2d = idx_km.reshape(bb * _K * nn // 128, 128)

    gf, gxn128 = _sc_gather([featr, xyz128], idx2d)
    gxn = gxn128[:, :16]

    rep = jnp.broadcast_to(
        xyz16.reshape(bb, 1, nn, 16), (bb, _K, nn, 16)
    ).reshape(bb, _K * nn, 16)

    w3a = jnp.pad(W1a[:, :3].T, ((0, 13), (0, 0)))
    wfa = W1a[:, 3:].T
    g1 = _mlp1(
        gf.reshape(bb, _K * nn, dd), gxn.reshape(bb, _K * nn, 16), rep,
        w3a, wfa, b1a.reshape(1, -1),
        W1b.T, b1b.reshape(1, -1), W1c.T, b1c.reshape(1, -1),
    )

    perm2d = perm_g.reshape(bb * nn // 128, 128)
    (ftab,) = _sc_gather([g1.reshape(bb * nn, dd)], perm2d)
    (g2f,) = _sc_gather([ftab], idx2d)

    w3b = jnp.pad(W2a[:, :3].T, ((0, 13), (0, 0)))
    wfb = W2a[:, 3:].T
    h = _mlp2(
        g2f.reshape(bb, _K * nn, dd), gxn.reshape(bb, _K * nn, 16), rep,
        w3b, wfb, b2a.reshape(1, -1),
    )

    (outr,) = _sc_gather([h.reshape(bb * nn, dd)], perm2d)
    return jnp.transpose(outr.reshape(bb, nn, dd), (0, 2, 1))


# speedup vs baseline: 1.0374x; 1.0374x over previous
import functools

import jax
import jax.numpy as jnp
from jax import lax
from jax.experimental import pallas as pl
from jax.experimental.pallas import tpu as pltpu
from jax.experimental.pallas import tpu_sc as plsc

_EPS = 1e-5
_K = 8


def _fps_body(pc_ref, perm_ref):
    x = pc_ref[0]
    y = pc_ref[1]
    z = pc_ref[2]
    bb, nn = x.shape
    lane = lax.broadcasted_iota(jnp.int32, (bb, nn), 1)
    boff = lax.broadcasted_iota(jnp.int32, (bb, nn), 0) * nn

    lane128 = lax.broadcasted_iota(jnp.int32, (bb, 128), 1)

    def body(j, st):
        dists, cperm = st
        far = jnp.argmax(dists, axis=1, keepdims=True).astype(jnp.int32)
        cperm = jnp.where(lane128 == j, far, cperm)
        sel = lane == far
        cx = jnp.sum(jnp.where(sel, x, 0.0), axis=1, keepdims=True)
        cy = jnp.sum(jnp.where(sel, y, 0.0), axis=1, keepdims=True)
        cz = jnp.sum(jnp.where(sel, z, 0.0), axis=1, keepdims=True)
        dx = x - cx
        dy = y - cy
        dz = z - cz
        d = dx * dx + dy * dy + dz * dz
        dists = jnp.minimum(dists, d)
        return dists, cperm

    def outer(c, st):
        dists, cperm = st
        dists, cperm = lax.fori_loop(0, 128, body, (dists, cperm))
        perm_ref[:, pl.ds(c * 128, 128)] = cperm
        return dists, cperm

    dists0 = (nn * 1.0 - lane.astype(jnp.float32)) * 1e6
    cperm0 = lane128 + lax.broadcasted_iota(jnp.int32, (bb, 128), 0)
    lax.fori_loop(0, nn // 128, outer, (dists0, cperm0))
    perm_ref[...] = perm_ref[...] + boff


def _fps(pc3, interpret=False):
    _, bb, nn = pc3.shape
    return pl.pallas_call(
        _fps_body,
        out_shape=jax.ShapeDtypeStruct((bb, nn), jnp.int32),
        interpret=interpret,
    )(pc3)


_RB = 256


def _knn_body(pc_ref, cen_ref, out_ref):
    b = pl.program_id(0)
    nn = pc_ref.shape[2]
    xyz = pc_ref[0]
    xj = xyz[0:1, :]
    yj = xyz[1:2, :]
    zj = xyz[2:3, :]
    cen = cen_ref[0]
    cx = cen[:, 0:1]
    cy = cen[:, 1:2]
    cz = cen[:, 2:3]
    dx = cx - xj
    dy = cy - yj
    dz = cz - zj
    d = dx * dx + dy * dy + dz * dz
    lane = lax.broadcasted_iota(jnp.int32, (d.shape[0], nn), 1)
    cols = []
    for _ in range(_K):
        m = jnp.min(d, axis=1, keepdims=True)
        j = jnp.min(jnp.where(d == m, lane, nn), axis=1, keepdims=True)
        cols.append(j)
        d = jnp.where(lane == j, jnp.inf, d)
    out_ref[0] = jnp.concatenate(cols, axis=1) + b * nn


def _knn(pc, xyzr, interpret=False):
    bb, _, nn = pc.shape
    return pl.pallas_call(
        _knn_body,
        grid=(bb, nn // _RB),
        in_specs=[
            pl.BlockSpec((1, 3, nn), lambda b, r: (b, 0, 0)),
            pl.BlockSpec((1, _RB, 3), lambda b, r: (b, r, 0)),
        ],
        out_specs=pl.BlockSpec((1, _RB, _K), lambda b, r: (b, r, 0)),
        out_shape=jax.ShapeDtypeStruct((bb, nn, _K), jnp.int32),
        interpret=interpret,
    )(pc, xyzr)


def _sc_gather(tables, idx2d):
    info = plsc.get_sparse_core_info()
    nw = info.num_cores * info.num_subcores
    nchunks_total, ch = idx2d.shape
    m = nchunks_total * ch
    nch = nchunks_total // nw
    ntab = len(tables)
    mesh = plsc.VectorSubcoreMesh(core_axis_name="c", subcore_axis_name="s")
    out_types = tuple(
        jax.ShapeDtypeStruct((m, t.shape[1]), t.dtype) for t in tables
    )
    scratch = [pltpu.VMEM((nch, ch), jnp.int32)]
    for t in tables:
        scratch.append(pltpu.VMEM((ch, t.shape[1]), t.dtype))
    scratch.append(pltpu.SemaphoreType.DMA)

    def body(*refs):
        tab_refs = refs[:ntab]
        idx_ref = refs[ntab]
        out_refs = refs[ntab + 1:2 * ntab + 1]
        idx_v = refs[2 * ntab + 1]
        row_bufs = refs[2 * ntab + 2:2 * ntab + 2 + ntab]
        sem = refs[-1]
        wid = lax.axis_index("s") * info.num_cores + lax.axis_index("c")
        base_chunk = wid * nch
        pltpu.sync_copy(idx_ref.at[pl.ds(base_chunk, nch)], idx_v)
        for c in range(nch):
            for t in range(ntab):
                pltpu.async_copy(
                    tab_refs[t].at[idx_v.at[c]], row_bufs[t], sem
                ).wait()
                pltpu.sync_copy(
                    row_bufs[t],
                    out_refs[t].at[pl.ds((base_chunk + c) * ch, ch)],
                )

    fn = pl.kernel(body, mesh=mesh, out_type=out_types, scratch_types=scratch)
    out = fn(*tables, idx2d)
    return out if isinstance(out, (tuple, list)) else (out,)


def _dot(a, b):
    return lax.dot_general(
        a, b, (((1,), (0,)), ((), ())), preferred_element_type=jnp.float32
    )


def _inorm(xx):
    m = jnp.mean(xx, axis=0, keepdims=True)
    c = xx - m
    v = jnp.mean(c * c, axis=0, keepdims=True)
    r = 1.0 / jnp.sqrt(v + _EPS)
    return c * r


def _maxpool_k(y, npts):
    m = y[0:npts, :]
    for k in range(1, _K):
        m = jnp.maximum(m, y[k * npts:(k + 1) * npts, :])
    return m


def _prep_body(ft_ref, xyz_ref, w3_ref, wf_ref, b_ref, y_ref, cp_ref):
    cp = _dot(xyz_ref[0], w3_ref[...])
    cp_ref[0] = cp
    y_ref[0] = _dot(ft_ref[0], wf_ref[...]) + cp + b_ref[...]


def _prep(ft, xyz16r, w3, wf, b, interpret=False):
    bb, npts, dd = ft.shape
    wspec = lambda shp: pl.BlockSpec(shp, lambda b: tuple(0 for _ in shp))
    return pl.pallas_call(
        _prep_body,
        grid=(bb,),
        in_specs=[
            pl.BlockSpec((1, npts, dd), lambda b: (b, 0, 0)),
            pl.BlockSpec((1, npts, 16), lambda b: (b, 0, 0)),
            wspec(w3.shape), wspec(wf.shape), wspec(b.shape),
        ],
        out_specs=[
            pl.BlockSpec((1, npts, dd), lambda b: (b, 0, 0)),
            pl.BlockSpec((1, npts, dd), lambda b: (b, 0, 0)),
        ],
        out_shape=[
            jax.ShapeDtypeStruct((bb, npts, dd), jnp.float32),
            jax.ShapeDtypeStruct((bb, npts, dd), jnp.float32),
        ],
        interpret=interpret,
    )(ft, xyz16r, w3, wf, b)


def _sub_center(g, cp):
    npts = cp.shape[0]
    return jnp.concatenate(
        [g[k * npts:(k + 1) * npts] - cp for k in range(_K)], axis=0
    )


def _mlp1_body(g_ref, cp_ref, w2_ref, b2_ref, w3c_ref, b3_ref, out_ref):
    y = _sub_center(g_ref[0], cp_ref[0])
    y = jnp.maximum(_inorm(y), 0.0)
    y = _dot(y, w2_ref[...]) + b2_ref[...]
    y = jnp.maximum(_inorm(y), 0.0)
    y = _dot(y, w3c_ref[...]) + b3_ref[...]
    y = jnp.maximum(_inorm(y), 0.0)
    out_ref[0] = _maxpool_k(y, out_ref.shape[1])


def _mlp1(g, cp, w2, b2, w3c, b3, interpret=False):
    bb, mk, dd = g.shape
    npts = mk // _K
    wspec = lambda shp: pl.BlockSpec(shp, lambda b: tuple(0 for _ in shp))
    return pl.pallas_call(
        _mlp1_body,
        grid=(bb,),
        in_specs=[
            pl.BlockSpec((1, mk, dd), lambda b: (b, 0, 0)),
            pl.BlockSpec((1, npts, dd), lambda b: (b, 0, 0)),
            wspec(w2.shape), wspec(b2.shape), wspec(w3c.shape), wspec(b3.shape),
        ],
        out_specs=pl.BlockSpec((1, npts, dd), lambda b: (b, 0, 0)),
        out_shape=jax.ShapeDtypeStruct((bb, npts, dd), jnp.float32),
        interpret=interpret,
    )(g, cp, w2, b2, w3c, b3)


def _mlp2_body(g_ref, cp_ref, out_ref):
    y = _inorm(_sub_center(g_ref[0], cp_ref[0]))
    out_ref[0] = _maxpool_k(y, out_ref.shape[1])


def _mlp2(g, cp, interpret=False):
    bb, mk, dd = g.shape
    npts = mk // _K
    return pl.pallas_call(
        _mlp2_body,
        grid=(bb,),
        in_specs=[
            pl.BlockSpec((1, mk, dd), lambda b: (b, 0, 0)),
            pl.BlockSpec((1, npts, dd), lambda b: (b, 0, 0)),
        ],
        out_specs=pl.BlockSpec((1, npts, dd), lambda b: (b, 0, 0)),
        out_shape=jax.ShapeDtypeStruct((bb, npts, dd), jnp.float32),
        interpret=interpret,
    )(g, cp)


def kernel(pc, feature, W1a, b1a, W1b, b1b, W1c, b1c, W2a, b2a):
    bb, _, nn = pc.shape
    dd = feature.shape[1]

    pc3 = jnp.transpose(pc, (1, 0, 2))
    xyzr = jnp.transpose(pc, (0, 2, 1))
    xyz16r = jnp.pad(xyzr, ((0, 0), (0, 0), (0, 13)))
    featr = jnp.transpose(feature, (0, 2, 1))

    perm_g = _fps(pc3)
    idx = _knn(pc, xyzr)
    idx_km = jnp.transpose(idx, (0, 2, 1))
    idx2d = idx_km.reshape(bb * _K * nn // 128, 128)
    perm2d = perm_g.reshape(bb * nn // 128, 128)

    w3a = jnp.pad(W1a[:, :3].T, ((0, 13), (0, 0)))
    wfa = W1a[:, 3:].T
    y1tab, cp1 = _prep(featr, xyz16r, w3a, wfa, b1a.reshape(1, -1))

    (g1g,) = _sc_gather([y1tab.reshape(bb * nn, dd)], idx2d)
    g1 = _mlp1(
        g1g.reshape(bb, _K * nn, dd), cp1,
        W1b.T, b1b.reshape(1, -1), W1c.T, b1c.reshape(1, -1),
    )

    (ftab,) = _sc_gather([g1.reshape(bb * nn, dd)], perm2d)

    w3b = jnp.pad(W2a[:, :3].T, ((0, 13), (0, 0)))
    wfb = W2a[:, 3:].T
    y2tab, cp2 = _prep(
        ftab.reshape(bb, nn, dd), xyz16r, w3b, wfb, b2a.reshape(1, -1)
    )

    (g2g,) = _sc_gather([y2tab.reshape(bb * nn, dd)], idx2d)
    h = _mlp2(g2g.reshape(bb, _K * nn, dd), cp2)

    (outr,) = _sc_gather([h.reshape(bb * nn, dd)], perm2d)
    return jnp.transpose(outr.reshape(bb, nn, dd), (0, 2, 1))


# speedup vs baseline: 1.1165x; 1.0762x over previous
import functools

import jax
import jax.numpy as jnp
from jax import lax
from jax.experimental import pallas as pl
from jax.experimental.pallas import tpu as pltpu
from jax.experimental.pallas import tpu_sc as plsc

_EPS = 1e-5
_K = 8


def _fps_body(pc_ref, perm_ref):
    x = pc_ref[0]
    y = pc_ref[1]
    z = pc_ref[2]
    bb, nn = x.shape
    lane = lax.broadcasted_iota(jnp.int32, (bb, nn), 1)
    boff = lax.broadcasted_iota(jnp.int32, (bb, nn), 0) * nn

    lane128 = lax.broadcasted_iota(jnp.int32, (bb, 128), 1)

    def body(j, st):
        dists, cperm = st
        far = jnp.argmax(dists, axis=1, keepdims=True).astype(jnp.int32)
        cperm = jnp.where(lane128 == j, far, cperm)
        sel = lane == far
        cx = jnp.sum(jnp.where(sel, x, 0.0), axis=1, keepdims=True)
        cy = jnp.sum(jnp.where(sel, y, 0.0), axis=1, keepdims=True)
        cz = jnp.sum(jnp.where(sel, z, 0.0), axis=1, keepdims=True)
        dx = x - cx
        dy = y - cy
        dz = z - cz
        d = dx * dx + dy * dy + dz * dz
        dists = jnp.minimum(dists, d)
        return dists, cperm

    def outer(c, st):
        dists, cperm = st
        dists, cperm = lax.fori_loop(0, 128, body, (dists, cperm))
        perm_ref[:, pl.ds(c * 128, 128)] = cperm
        return dists, cperm

    dists0 = (nn * 1.0 - lane.astype(jnp.float32)) * 1e6
    cperm0 = lane128 + lax.broadcasted_iota(jnp.int32, (bb, 128), 0)
    lax.fori_loop(0, nn // 128, outer, (dists0, cperm0))
    perm_ref[...] = perm_ref[...] + boff


def _fps(pc3, interpret=False):
    _, bb, nn = pc3.shape
    return pl.pallas_call(
        _fps_body,
        out_shape=jax.ShapeDtypeStruct((bb, nn), jnp.int32),
        interpret=interpret,
    )(pc3)


_RB = 256


def _knn_body(pc_ref, cen_ref, out_ref):
    b = pl.program_id(0)
    nn = pc_ref.shape[2]
    xyz = pc_ref[0]
    xj = xyz[0:1, :]
    yj = xyz[1:2, :]
    zj = xyz[2:3, :]
    cen = cen_ref[0]
    cx = cen[:, 0:1]
    cy = cen[:, 1:2]
    cz = cen[:, 2:3]
    dx = cx - xj
    dy = cy - yj
    dz = cz - zj
    d = dx * dx + dy * dy + dz * dz
    lane = lax.broadcasted_iota(jnp.int32, (d.shape[0], nn), 1)
    cols = []
    for _ in range(_K):
        j = jnp.argmin(d, axis=1, keepdims=True).astype(jnp.int32)
        cols.append(j)
        d = jnp.where(lane == j, jnp.inf, d)
    out_ref[0] = jnp.concatenate(cols, axis=1) + b * nn


def _knn(pc, xyzr, interpret=False):
    bb, _, nn = pc.shape
    return pl.pallas_call(
        _knn_body,
        grid=(bb, nn // _RB),
        in_specs=[
            pl.BlockSpec((1, 3, nn), lambda b, r: (b, 0, 0)),
            pl.BlockSpec((1, _RB, 3), lambda b, r: (b, r, 0)),
        ],
        out_specs=pl.BlockSpec((1, _RB, _K), lambda b, r: (b, r, 0)),
        out_shape=jax.ShapeDtypeStruct((bb, nn, _K), jnp.int32),
        interpret=interpret,
    )(pc, xyzr)


def _sc_gather(tables, idx2d):
    info = plsc.get_sparse_core_info()
    nw = info.num_cores * info.num_subcores
    nchunks_total, ch = idx2d.shape
    m = nchunks_total * ch
    nch = nchunks_total // nw
    ntab = len(tables)
    mesh = plsc.VectorSubcoreMesh(core_axis_name="c", subcore_axis_name="s")
    out_types = tuple(
        jax.ShapeDtypeStruct((m, t.shape[1]), t.dtype) for t in tables
    )
    scratch = [pltpu.VMEM((nch, ch), jnp.int32)]
    for t in tables:
        scratch.append(pltpu.VMEM((ch, t.shape[1]), t.dtype))
    scratch.append(pltpu.SemaphoreType.DMA)

    def body(*refs):
        tab_refs = refs[:ntab]
        idx_ref = refs[ntab]
        out_refs = refs[ntab + 1:2 * ntab + 1]
        idx_v = refs[2 * ntab + 1]
        row_bufs = refs[2 * ntab + 2:2 * ntab + 2 + ntab]
        sem = refs[-1]
        wid = lax.axis_index("s") * info.num_cores + lax.axis_index("c")
        base_chunk = wid * nch
        pltpu.sync_copy(idx_ref.at[pl.ds(base_chunk, nch)], idx_v)
        for c in range(nch):
            for t in range(ntab):
                pltpu.async_copy(
                    tab_refs[t].at[idx_v.at[c]], row_bufs[t], sem
                ).wait()
                pltpu.sync_copy(
                    row_bufs[t],
                    out_refs[t].at[pl.ds((base_chunk + c) * ch, ch)],
                )

    fn = pl.kernel(body, mesh=mesh, out_type=out_types, scratch_types=scratch)
    out = fn(*tables, idx2d)
    return out if isinstance(out, (tuple, list)) else (out,)


def _dot(a, b):
    return lax.dot_general(
        a, b, (((1,), (0,)), ((), ())), preferred_element_type=jnp.float32
    )


def _inorm_stats(xx):
    n = xx.shape[0]
    s1 = jnp.sum(xx, axis=0, keepdims=True) * (1.0 / n)
    s2 = jnp.sum(xx * xx, axis=0, keepdims=True) * (1.0 / n)
    r = 1.0 / jnp.sqrt(jnp.maximum(s2 - s1 * s1, 0.0) + _EPS)
    return s1, r


def _maxpool_k(y, npts):
    m = y[0:npts, :]
    for k in range(1, _K):
        m = jnp.maximum(m, y[k * npts:(k + 1) * npts, :])
    return m


def _prep_body(ft_ref, xyz_ref, w3_ref, wf_ref, b_ref, y_ref, cp_ref):
    cp = _dot(xyz_ref[0], w3_ref[...])
    cp_ref[0] = cp
    y_ref[0] = _dot(ft_ref[0], wf_ref[...]) + cp + b_ref[...]


def _prep(ft, xyz16r, w3, wf, b, interpret=False):
    bb, npts, dd = ft.shape
    wspec = lambda shp: pl.BlockSpec(shp, lambda b: tuple(0 for _ in shp))
    return pl.pallas_call(
        _prep_body,
        grid=(bb,),
        in_specs=[
            pl.BlockSpec((1, npts, dd), lambda b: (b, 0, 0)),
            pl.BlockSpec((1, npts, 16), lambda b: (b, 0, 0)),
            wspec(w3.shape), wspec(wf.shape), wspec(b.shape),
        ],
        out_specs=[
            pl.BlockSpec((1, npts, dd), lambda b: (b, 0, 0)),
            pl.BlockSpec((1, npts, dd), lambda b: (b, 0, 0)),
        ],
        out_shape=[
            jax.ShapeDtypeStruct((bb, npts, dd), jnp.float32),
            jax.ShapeDtypeStruct((bb, npts, dd), jnp.float32),
        ],
        interpret=interpret,
    )(ft, xyz16r, w3, wf, b)


def _sub_center(g, cp):
    npts = cp.shape[0]
    return jnp.concatenate(
        [g[k * npts:(k + 1) * npts] - cp for k in range(_K)], axis=0
    )


def _mlp1_body(g_ref, cp_ref, w2_ref, b2_ref, w3c_ref, b3_ref, out_ref):
    y = _sub_center(g_ref[0], cp_ref[0])
    m1, r1 = _inorm_stats(y)
    y = jnp.maximum(y - m1, 0.0)
    w2s = w2_ref[...] * jnp.transpose(r1)
    y = _dot(y, w2s) + b2_ref[...]
    m2, r2 = _inorm_stats(y)
    y = jnp.maximum(y - m2, 0.0)
    w3s = w3c_ref[...] * jnp.transpose(r2)
    y = _dot(y, w3s) + b3_ref[...]
    m3, r3 = _inorm_stats(y)
    out_ref[0] = _maxpool_k(jnp.maximum(y - m3, 0.0), out_ref.shape[1]) * r3


def _mlp1(g, cp, w2, b2, w3c, b3, interpret=False):
    bb, mk, dd = g.shape
    npts = mk // _K
    wspec = lambda shp: pl.BlockSpec(shp, lambda b: tuple(0 for _ in shp))
    return pl.pallas_call(
        _mlp1_body,
        grid=(bb,),
        in_specs=[
            pl.BlockSpec((1, mk, dd), lambda b: (b, 0, 0)),
            pl.BlockSpec((1, npts, dd), lambda b: (b, 0, 0)),
            wspec(w2.shape), wspec(b2.shape), wspec(w3c.shape), wspec(b3.shape),
        ],
        out_specs=pl.BlockSpec((1, npts, dd), lambda b: (b, 0, 0)),
        out_shape=jax.ShapeDtypeStruct((bb, npts, dd), jnp.float32),
        interpret=interpret,
    )(g, cp, w2, b2, w3c, b3)


def _mlp2_body(g_ref, cp_ref, out_ref):
    y = _sub_center(g_ref[0], cp_ref[0])
    m1, r1 = _inorm_stats(y)
    out_ref[0] = _maxpool_k(y - m1, out_ref.shape[1]) * r1


def _mlp2(g, cp, interpret=False):
    bb, mk, dd = g.shape
    npts = mk // _K
    return pl.pallas_call(
        _mlp2_body,
        grid=(bb,),
        in_specs=[
            pl.BlockSpec((1, mk, dd), lambda b: (b, 0, 0)),
            pl.BlockSpec((1, npts, dd), lambda b: (b, 0, 0)),
        ],
        out_specs=pl.BlockSpec((1, npts, dd), lambda b: (b, 0, 0)),
        out_shape=jax.ShapeDtypeStruct((bb, npts, dd), jnp.float32),
        interpret=interpret,
    )(g, cp)


def kernel(pc, feature, W1a, b1a, W1b, b1b, W1c, b1c, W2a, b2a):
    bb, _, nn = pc.shape
    dd = feature.shape[1]

    pc3 = jnp.transpose(pc, (1, 0, 2))
    xyzr = jnp.transpose(pc, (0, 2, 1))
    xyz16r = jnp.pad(xyzr, ((0, 0), (0, 0), (0, 13)))
    featr = jnp.transpose(feature, (0, 2, 1))

    perm_g = _fps(pc3)
    idx = _knn(pc, xyzr)
    idx_km = jnp.transpose(idx, (0, 2, 1))
    idx2d = idx_km.reshape(bb * _K * nn // 128, 128)
    perm2d = perm_g.reshape(bb * nn // 128, 128)

    w3a = jnp.pad(W1a[:, :3].T, ((0, 13), (0, 0)))
    wfa = W1a[:, 3:].T
    y1tab, cp1 = _prep(featr, xyz16r, w3a, wfa, b1a.reshape(1, -1))

    (g1g,) = _sc_gather([y1tab.reshape(bb * nn, dd)], idx2d)
    g1 = _mlp1(
        g1g.reshape(bb, _K * nn, dd), cp1,
        W1b.T, b1b.reshape(1, -1), W1c.T, b1c.reshape(1, -1),
    )

    (ftab,) = _sc_gather([g1.reshape(bb * nn, dd)], perm2d)

    w3b = jnp.pad(W2a[:, :3].T, ((0, 13), (0, 0)))
    wfb = W2a[:, 3:].T
    y2tab, cp2 = _prep(
        ftab.reshape(bb, nn, dd), xyz16r, w3b, wfb, b2a.reshape(1, -1)
    )

    (g2g,) = _sc_gather([y2tab.reshape(bb * nn, dd)], idx2d)
    h = _mlp2(g2g.reshape(bb, _K * nn, dd), cp2)

    (outr,) = _sc_gather([h.reshape(bb * nn, dd)], perm2d)
    return jnp.transpose(outr.reshape(bb, nn, dd), (0, 2, 1))
